# baseline (speedup 1.0000x reference)
import functools

import jax
import jax.numpy as jnp
from jax import lax
from jax.experimental import pallas as pl
from jax.experimental.pallas import tpu as pltpu
from jax.experimental.pallas import tpu_sc as plsc

NBUF = 16


@functools.lru_cache(maxsize=None)
def _make_gather(batch, dim):
    info = plsc.get_sparse_core_info()
    nw = info.num_cores * info.num_subcores
    b_per_w = batch // nw
    n_grp = b_per_w // NBUF
    mesh = plsc.VectorSubcoreMesh(core_axis_name="c", subcore_axis_name="s")

    @functools.partial(
        pl.kernel,
        mesh=mesh,
        compiler_params=pltpu.CompilerParams(use_tc_tiling_on_sc=True, needs_layout_passes=False),
        out_type=jax.ShapeDtypeStruct((dim, batch), jnp.float32),
        scratch_types=[
            pltpu.VMEM((b_per_w,), jnp.int32),
            pltpu.VMEM((3, NBUF, dim, 128), jnp.float32),
            pltpu.VMEM((dim, b_per_w), jnp.float32),
            pltpu.SemaphoreType.DMA,
        ],
    )
    def gather_kernel(idx_hbm, table_t_hbm, out_t_hbm, idx_v, ring_v, stage_v, sem):
        wid = lax.axis_index("s") * info.num_cores + lax.axis_index("c")
        base = wid * b_per_w
        pltpu.sync_copy(idx_hbm.at[pl.ds(base, b_per_w)], idx_v)
        lanes = lax.iota(jnp.int32, 16) % dim

        def fire(vec, buf):
            for u in range(NBUF):
                t = pl.multiple_of(vec[u] & -128, 128)
                pltpu.async_copy(
                    table_t_hbm.at[:, pl.ds(t, 128)], ring_v.at[buf, u], sem
                )

        def drain(g, vec, buf):
            i0 = g * NBUF
            for u in range(NBUF):
                pltpu.make_async_copy(
                    table_t_hbm.at[:, pl.ds(0, 128)], ring_v.at[buf, u], sem
                ).wait()
                col = plsc.load_gather(
                    ring_v.at[buf, u],
                    [lanes, jnp.full((16,), vec[u] & 127, jnp.int32)],
                )
                plsc.store_scatter(
                    stage_v, [lanes, jnp.full((16,), i0 + u, jnp.int32)], col
                )

        vec0 = idx_v[pl.ds(0, NBUF)]
        vec1 = idx_v[pl.ds(NBUF, NBUF)]
        fire(vec0, 0)
        fire(vec1, 1)

        def group(g, carry):
            vec_cur, vec_nxt = carry
            vec_n2 = idx_v[pl.ds((g + 2) * NBUF, NBUF)]
            fire(vec_n2, lax.rem(g + 2, 3))
            drain(g, vec_cur, lax.rem(g, 3))
            return (vec_nxt, vec_n2)

        vec_a, vec_b = lax.fori_loop(
            0, n_grp - 2, group, (vec0, vec1), unroll=False
        )
        drain(n_grp - 2, vec_a, lax.rem(n_grp - 2, 3))
        drain(n_grp - 1, vec_b, lax.rem(n_grp - 1, 3))
        pltpu.sync_copy(stage_v, out_t_hbm.at[:, pl.ds(base, b_per_w)])

    return gather_kernel


def kernel(element, table):
    batch = element.shape[0]
    dim = table.shape[1]
    idx = element.astype(jnp.int32)
    out_t = _make_gather(batch, dim)(idx, table.T)
    return out_t.T


# speedup vs baseline: 1.0264x; 1.0264x over previous
import functools

import jax
import jax.numpy as jnp
from jax import lax
from jax.experimental import pallas as pl
from jax.experimental.pallas import tpu as pltpu
from jax.experimental.pallas import tpu_sc as plsc

NBUF = 16


@functools.lru_cache(maxsize=None)
def _make_gather(batch, dim):
    info = plsc.get_sparse_core_info()
    nw = info.num_cores * info.num_subcores
    b_per_w = batch // nw
    n_grp = b_per_w // NBUF
    mesh = plsc.VectorSubcoreMesh(core_axis_name="c", subcore_axis_name="s")

    @functools.partial(
        pl.kernel,
        mesh=mesh,
        compiler_params=pltpu.CompilerParams(use_tc_tiling_on_sc=True, needs_layout_passes=False),
        out_type=jax.ShapeDtypeStruct((dim, batch), jnp.float32),
        scratch_types=[
            pltpu.VMEM((b_per_w,), jnp.int32),
            pltpu.VMEM((2, NBUF, dim, 128), jnp.float32),
            pltpu.VMEM((dim, b_per_w), jnp.float32),
            pltpu.SemaphoreType.DMA,
        ],
    )
    def gather_kernel(idx_hbm, table_t_hbm, out_t_hbm, idx_v, ring_v, stage_v, sem):
        wid = lax.axis_index("s") * info.num_cores + lax.axis_index("c")
        base = wid * b_per_w
        pltpu.sync_copy(idx_hbm.at[pl.ds(base, b_per_w)], idx_v)
        lanes = lax.iota(jnp.int32, 16) % dim

        def fire(vec, buf):
            for u in range(NBUF):
                t = pl.multiple_of(vec[u] & -128, 128)
                pltpu.async_copy(
                    table_t_hbm.at[:, pl.ds(t, 128)], ring_v.at[buf, u], sem
                )

        def drain(g, vec, buf):
            i0 = g * NBUF
            for u in range(NBUF):
                pltpu.make_async_copy(
                    table_t_hbm.at[:, pl.ds(0, 128)], ring_v.at[buf, u], sem
                ).wait()
                col = plsc.load_gather(
                    ring_v.at[buf, u],
                    [lanes, jnp.full((16,), vec[u] & 127, jnp.int32)],
                )
                plsc.store_scatter(
                    stage_v, [lanes, jnp.full((16,), i0 + u, jnp.int32)], col
                )

        vec0 = idx_v[pl.ds(0, NBUF)]
        fire(vec0, 0)

        def group(g, vec_cur):
            vec_next = idx_v[pl.ds((g + 1) * NBUF, NBUF)]
            fire(vec_next, lax.rem(g + 1, 2))
            drain(g, vec_cur, lax.rem(g, 2))
            return vec_next

        vec_last = lax.fori_loop(0, n_grp - 1, group, vec0, unroll=False)
        drain(n_grp - 1, vec_last, lax.rem(n_grp - 1, 2))
        pltpu.sync_copy(stage_v, out_t_hbm.at[:, pl.ds(base, b_per_w)])

    return gather_kernel


def kernel(element, table):
    batch = element.shape[0]
    dim = table.shape[1]
    idx = element.astype(jnp.int32)
    out_t = _make_gather(batch, dim)(idx, table.T)
    return out_t.T
